# Initial kernel scaffold; baseline (speedup 1.0000x reference)
import jax
import jax.numpy as jnp
from jax.experimental import pallas as pl


def kernel(x, edge_index, edge_attr, W0, b0, bn_gamma, bn_beta, bn_mean, bn_var, W1, b1):
    raise NotImplementedError("write your pallas kernel here")


# speedup vs baseline: 3.4071x; 3.4071x over previous
import functools

import jax
import jax.numpy as jnp
from jax import lax
from jax.experimental import pallas as pl
from jax.experimental.pallas import tpu as pltpu
from jax.experimental.pallas import tpu_sc as plsc

NC = 2
NS = 16
LANES = 16
BN_EPS_ = 1e-5


def _mm_ew_body(x_ref, w_ref, a_ref, o_ref, ew_ref):
    o_ref[...] = jnp.dot(x_ref[...], w_ref[...],
                         preferred_element_type=jnp.float32)
    lane = lax.broadcasted_iota(jnp.int32, (128, 32), 0)
    grp = lax.broadcasted_iota(jnp.int32, (128, 32), 1)
    sel = jnp.where(lane // 4 == grp, 0.25, 0.0).astype(jnp.float32)
    ew_ref[...] = jnp.dot(a_ref[...], sel,
                          preferred_element_type=jnp.float32)


def _matmul_ew(x, w, attr2d, block_rows=1000):
    n, k = x.shape
    _, m = w.shape
    na = attr2d.shape[0]
    assert na == n
    grid = n // block_rows
    return pl.pallas_call(
        _mm_ew_body,
        grid=(grid,),
        in_specs=[
            pl.BlockSpec((block_rows, k), lambda i: (i, 0)),
            pl.BlockSpec((k, m), lambda i: (0, 0)),
            pl.BlockSpec((block_rows, 128), lambda i: (i, 0)),
        ],
        out_specs=[
            pl.BlockSpec((block_rows, m), lambda i: (i, 0)),
            pl.BlockSpec((block_rows, 32), lambda i: (i, 0)),
        ],
        out_shape=[
            jax.ShapeDtypeStruct((n, m), jnp.float32),
            jax.ShapeDtypeStruct((na, 32), jnp.float32),
        ],
    )(x, w, attr2d)


def _bn_relu_body(p0_ref, p1_ref, b_ref, g_ref, be_ref, mu_ref, var_ref,
                  o_ref):
    scale = g_ref[...] * lax.rsqrt(var_ref[...] + BN_EPS_)
    shift = be_ref[...] - mu_ref[...] * scale + b_ref[...] * scale
    agg = p0_ref[...] + p1_ref[...]
    o_ref[...] = jnp.maximum(agg * scale + shift, 0.0)


def _bn_relu(p0, p1, b0, gamma, beta, mean, var, block_rows=1000):
    n, d = p0.shape
    vec = lambda a: a.reshape(1, d)
    vspec = pl.BlockSpec((1, d), lambda i: (0, 0))
    bspec = pl.BlockSpec((block_rows, d), lambda i: (i, 0))
    return pl.pallas_call(
        _bn_relu_body,
        grid=(n // block_rows,),
        in_specs=[bspec, bspec, vspec, vspec, vspec, vspec, vspec],
        out_specs=bspec,
        out_shape=jax.ShapeDtypeStruct((n, d), jnp.float32),
    )(p0, p1, vec(b0), vec(gamma), vec(beta), vec(mean), vec(var))


def _mm_bias_body(q0_ref, q1_ref, w_ref, b_ref, o_ref):
    agg = q0_ref[...] + q1_ref[...]
    o_ref[...] = jnp.dot(agg, w_ref[...],
                         preferred_element_type=jnp.float32) + b_ref[...]


def _matmul_bias(q0, q1, w, b, block_rows=1000):
    n, k = q0.shape
    _, m = w.shape
    bspec = pl.BlockSpec((block_rows, k), lambda i: (i, 0))
    return pl.pallas_call(
        _mm_bias_body,
        grid=(n // block_rows,),
        in_specs=[
            bspec, bspec,
            pl.BlockSpec((k, m), lambda i: (0, 0)),
            pl.BlockSpec((1, m), lambda i: (0, 0)),
        ],
        out_specs=pl.BlockSpec((block_rows, m), lambda i: (i, 0)),
        out_shape=jax.ShapeDtypeStruct((n, m), jnp.float32),
    )(q0, q1, w, b.reshape(1, m))


def _make_sc_agg(weighted, n, e, d, chunk):
    per_tile = e // (NC * NS)
    assert per_tile * NC * NS == e
    n_chunks = per_tile // chunk
    assert n_chunks * chunk == per_tile
    main_rows = (n // NS) // 8 * 8
    tail_rows = n - main_rows * NS
    assert 0 <= tail_rows and tail_rows % 8 == 0

    mesh = plsc.VectorSubcoreMesh(core_axis_name="c", subcore_axis_name="s")
    out_sds = jax.ShapeDtypeStruct((n, d), jnp.float32)

    scratch = [
        pltpu.VMEM_SHARED((n, d), jnp.float32),
        pltpu.VMEM((chunk,), jnp.int32),
        pltpu.VMEM((chunk,), jnp.int32),
        pltpu.VMEM((chunk, d), jnp.float32),
        pltpu.SemaphoreType.DMA,
    ]
    if weighted:
        scratch.append(pltpu.VMEM((chunk + LANES,), jnp.float32))

    @functools.partial(
        pl.kernel, mesh=mesh,
        out_type=(out_sds, out_sds),
        scratch_types=scratch,
        compiler_params=pltpu.CompilerParams(use_tc_tiling_on_sc=False),
    )
    def sc_agg(h_hbm, src_hbm, dst_hbm, attr_hbm, zero_hbm,
               out0, out1, acc_sh, src_v, dst_v, rows_v, sem, *maybe_attr):
        cid = lax.axis_index("c")
        sid = lax.axis_index("s")
        wid = sid * NC + cid
        base0 = wid * per_tile
        row0 = pl.multiple_of(sid * main_rows, 8)

        pltpu.sync_copy(zero_hbm.at[pl.ds(row0, main_rows)],
                        acc_sh.at[pl.ds(row0, main_rows)])
        if tail_rows:
            @pl.when(sid == NS - 1)
            def _():
                pltpu.sync_copy(zero_hbm.at[pl.ds(NS * main_rows, tail_rows)],
                                acc_sh.at[pl.ds(NS * main_rows, tail_rows)])
        plsc.subcore_barrier()

        def chunk_body(i, carry):
            base = base0 + i * chunk
            pltpu.sync_copy(src_hbm.at[pl.ds(base, chunk)], src_v)
            pltpu.sync_copy(dst_hbm.at[pl.ds(base, chunk)], dst_v)
            if weighted:
                ew_v = maybe_attr[0]
                pltpu.sync_copy(attr_hbm.at[pl.ds(base, chunk)],
                                ew_v.at[pl.ds(0, chunk)])
            pltpu.async_copy(h_hbm.at[src_v], rows_v, sem).wait()
            if weighted:
                def edge_body(ei, c2):
                    w = ew_v[pl.ds(ei, LANES)][0]
                    for q in range(d // LANES):
                        sl = pl.ds(q * LANES, LANES)
                        rows_v[ei, sl] = rows_v[ei, sl] * w
                    return c2
                lax.fori_loop(0, chunk, edge_body, 0)
            pltpu.sync_copy(rows_v, acc_sh.at[dst_v], add=True)
            return carry
        lax.fori_loop(0, n_chunks, chunk_body, 0)

        plsc.subcore_barrier()

        @pl.when(cid == 0)
        def _():
            pltpu.sync_copy(acc_sh.at[pl.ds(row0, main_rows)],
                            out0.at[pl.ds(row0, main_rows)])
            if tail_rows:
                @pl.when(sid == NS - 1)
                def _():
                    pltpu.sync_copy(
                        acc_sh.at[pl.ds(NS * main_rows, tail_rows)],
                        out0.at[pl.ds(NS * main_rows, tail_rows)])

        @pl.when(cid == 1)
        def _():
            pltpu.sync_copy(acc_sh.at[pl.ds(row0, main_rows)],
                            out1.at[pl.ds(row0, main_rows)])
            if tail_rows:
                @pl.when(sid == NS - 1)
                def _():
                    pltpu.sync_copy(
                        acc_sh.at[pl.ds(NS * main_rows, tail_rows)],
                        out1.at[pl.ds(NS * main_rows, tail_rows)])

    return sc_agg


def kernel(x, edge_index, edge_attr, W0, b0, bn_gamma, bn_beta, bn_mean,
           bn_var, W1, b1):
    n, din = x.shape
    e = edge_index.shape[1]
    h_dim = W0.shape[1]
    src = edge_index[0]
    dst = edge_index[1]
    zeros = jnp.zeros((n, h_dim), jnp.float32)

    attr2d = edge_attr.astype(jnp.float32).reshape(e // 32, 128)
    h, ew2d = _matmul_ew(x.astype(jnp.float32), W0, attr2d,
                         block_rows=n // 10)
    ew = ew2d.reshape(-1)
    agg_w = _make_sc_agg(True, n, e, h_dim, 80)
    p0, p1 = agg_w(h, src, dst, ew, zeros)
    h1 = _bn_relu(p0, p1, b0, bn_gamma, bn_beta, bn_mean, bn_var)
    agg_p = _make_sc_agg(False, n, e, h_dim, 80)
    q0, q1 = agg_p(h1, src, dst, ew, zeros)
    return _matmul_bias(q0, q1, W1, b1)


# speedup vs baseline: 4.9604x; 1.4559x over previous
import functools

import jax
import jax.numpy as jnp
from jax import lax
from jax.experimental import pallas as pl
from jax.experimental.pallas import tpu as pltpu
from jax.experimental.pallas import tpu_sc as plsc

NC = 2
NS = 16
LANES = 16
BN_EPS_ = 1e-5


def _mm_ew_body(x_ref, w_ref, a_ref, o_ref, ew_ref):
    o_ref[...] = jnp.dot(x_ref[...], w_ref[...],
                         preferred_element_type=jnp.float32)
    lane = lax.broadcasted_iota(jnp.int32, (128, 32), 0)
    grp = lax.broadcasted_iota(jnp.int32, (128, 32), 1)
    sel = jnp.where(lane // 4 == grp, 0.25, 0.0).astype(jnp.float32)
    ew_ref[...] = jnp.dot(a_ref[...], sel,
                          preferred_element_type=jnp.float32)


def _matmul_ew(x, w, attr2d, block_rows=1000):
    n, k = x.shape
    _, m = w.shape
    na = attr2d.shape[0]
    assert na == n
    grid = n // block_rows
    return pl.pallas_call(
        _mm_ew_body,
        grid=(grid,),
        in_specs=[
            pl.BlockSpec((block_rows, k), lambda i: (i, 0)),
            pl.BlockSpec((k, m), lambda i: (0, 0)),
            pl.BlockSpec((block_rows, 128), lambda i: (i, 0)),
        ],
        out_specs=[
            pl.BlockSpec((block_rows, m), lambda i: (i, 0)),
            pl.BlockSpec((block_rows, 32), lambda i: (i, 0)),
        ],
        out_shape=[
            jax.ShapeDtypeStruct((n, m), jnp.float32),
            jax.ShapeDtypeStruct((na, 32), jnp.float32),
        ],
    )(x, w, attr2d)


def _bn_relu_body(p0_ref, p1_ref, b_ref, g_ref, be_ref, mu_ref, var_ref,
                  o_ref):
    scale = g_ref[...] * lax.rsqrt(var_ref[...] + BN_EPS_)
    shift = be_ref[...] - mu_ref[...] * scale + b_ref[...] * scale
    agg = p0_ref[...] + p1_ref[...]
    o_ref[...] = jnp.maximum(agg * scale + shift, 0.0)


def _bn_relu(p0, p1, b0, gamma, beta, mean, var, block_rows=1000):
    n, d = p0.shape
    vec = lambda a: a.reshape(1, d)
    vspec = pl.BlockSpec((1, d), lambda i: (0, 0))
    bspec = pl.BlockSpec((block_rows, d), lambda i: (i, 0))
    return pl.pallas_call(
        _bn_relu_body,
        grid=(n // block_rows,),
        in_specs=[bspec, bspec, vspec, vspec, vspec, vspec, vspec],
        out_specs=bspec,
        out_shape=jax.ShapeDtypeStruct((n, d), jnp.float32),
    )(p0, p1, vec(b0), vec(gamma), vec(beta), vec(mean), vec(var))


def _mm_bias_body(q0_ref, q1_ref, w_ref, b_ref, o_ref):
    agg = q0_ref[...] + q1_ref[...]
    o_ref[...] = jnp.dot(agg, w_ref[...],
                         preferred_element_type=jnp.float32) + b_ref[...]


def _matmul_bias(q0, q1, w, b, block_rows=1000):
    n, k = q0.shape
    _, m = w.shape
    bspec = pl.BlockSpec((block_rows, k), lambda i: (i, 0))
    return pl.pallas_call(
        _mm_bias_body,
        grid=(n // block_rows,),
        in_specs=[
            bspec, bspec,
            pl.BlockSpec((k, m), lambda i: (0, 0)),
            pl.BlockSpec((1, m), lambda i: (0, 0)),
        ],
        out_specs=pl.BlockSpec((block_rows, m), lambda i: (i, 0)),
        out_shape=jax.ShapeDtypeStruct((n, m), jnp.float32),
    )(q0, q1, w, b.reshape(1, m))


def _make_sc_agg(weighted, n, e, d, chunk):
    per_tile = e // (NC * NS)
    assert per_tile * NC * NS == e
    n_chunks = per_tile // chunk
    n_pairs = n_chunks // 2
    assert n_chunks * chunk == per_tile and n_pairs * 2 == n_chunks
    assert chunk <= 128
    main_rows = (n // NS) // 8 * 8
    tail_rows = n - main_rows * NS
    assert 0 <= tail_rows and tail_rows % 8 == 0

    mesh = plsc.VectorSubcoreMesh(core_axis_name="c", subcore_axis_name="s")
    out_sds = jax.ShapeDtypeStruct((n, d), jnp.float32)

    scratch = [
        pltpu.VMEM_SHARED((n, d), jnp.float32),
        pltpu.VMEM((n_chunks, chunk), jnp.int32),
        pltpu.VMEM((n_chunks, chunk), jnp.int32),
        pltpu.VMEM((chunk, d), jnp.float32),
        pltpu.VMEM((chunk, d), jnp.float32),
        pltpu.SemaphoreType.DMA,
        pltpu.SemaphoreType.DMA,
        pltpu.SemaphoreType.DMA,
        pltpu.SemaphoreType.DMA,
    ]
    if weighted:
        scratch.append(pltpu.VMEM((per_tile + LANES,), jnp.float32))

    @functools.partial(
        pl.kernel, mesh=mesh,
        out_type=(out_sds, out_sds),
        scratch_types=scratch,
        compiler_params=pltpu.CompilerParams(use_tc_tiling_on_sc=False),
    )
    def sc_agg(h_hbm, src_hbm, dst_hbm, attr_hbm, zero_hbm,
               out0, out1, acc_sh, srcb, dstb, rows0, rows1,
               g0, g1, s0, s1, *maybe_ew):
        cid = lax.axis_index("c")
        sid = lax.axis_index("s")
        wid = sid * NC + cid
        row0 = pl.multiple_of(sid * main_rows, 8)
        rowsb = (rows0, rows1)
        gsem = (g0, g1)
        ssem = (s0, s1)

        pltpu.sync_copy(src_hbm.at[pl.ds(wid * n_chunks, n_chunks)], srcb)
        pltpu.sync_copy(dst_hbm.at[pl.ds(wid * n_chunks, n_chunks)], dstb)
        if weighted:
            ew_v = maybe_ew[0]
            pltpu.sync_copy(attr_hbm.at[pl.ds(wid * per_tile, per_tile)],
                            ew_v.at[pl.ds(0, per_tile)])

        pltpu.sync_copy(zero_hbm.at[pl.ds(row0, main_rows)],
                        acc_sh.at[pl.ds(row0, main_rows)])
        if tail_rows:
            @pl.when(sid == NS - 1)
            def _():
                pltpu.sync_copy(zero_hbm.at[pl.ds(NS * main_rows, tail_rows)],
                                acc_sh.at[pl.ds(NS * main_rows, tail_rows)])

        pltpu.async_copy(h_hbm.at[srcb.at[0]], rows0, g0)
        plsc.subcore_barrier()

        def scale_rows(ci, rv):
            def edge_body(ei, c2):
                w = ew_v[pl.ds(ci * chunk + ei, LANES)][0]
                for q in range(d // LANES):
                    sl = pl.ds(q * LANES, LANES)
                    rv[ei, sl] = rv[ei, sl] * w
                return c2
            lax.fori_loop(0, chunk, edge_body, 0)

        def pair_body(i, carry):
            for b in (0, 1):
                ci = 2 * i + b
                rv, nrv = rowsb[b], rowsb[1 - b]
                pltpu.make_async_copy(h_hbm.at[srcb.at[ci]], rv,
                                      gsem[b]).wait()
                if b == 0:
                    @pl.when(i >= 1)
                    def _():
                        pltpu.make_async_copy(
                            nrv, acc_sh.at[dstb.at[ci - 1]],
                            ssem[1 - b]).wait()
                    pltpu.async_copy(h_hbm.at[srcb.at[ci + 1]], nrv,
                                     gsem[1 - b])
                else:
                    pltpu.make_async_copy(nrv, acc_sh.at[dstb.at[ci - 1]],
                                          ssem[1 - b]).wait()

                    @pl.when(i < n_pairs - 1)
                    def _():
                        pltpu.async_copy(h_hbm.at[srcb.at[ci + 1]], nrv,
                                         gsem[1 - b])
                if weighted:
                    scale_rows(ci, rv)
                pltpu.async_copy(rv, acc_sh.at[dstb.at[ci]], ssem[b],
                                 add=True)
            return carry
        lax.fori_loop(0, n_pairs, pair_body, 0)
        pltpu.make_async_copy(rows1, acc_sh.at[dstb.at[n_chunks - 1]],
                              s1).wait()

        plsc.subcore_barrier()

        @pl.when(cid == 0)
        def _():
            pltpu.sync_copy(acc_sh.at[pl.ds(row0, main_rows)],
                            out0.at[pl.ds(row0, main_rows)])
            if tail_rows:
                @pl.when(sid == NS - 1)
                def _():
                    pltpu.sync_copy(
                        acc_sh.at[pl.ds(NS * main_rows, tail_rows)],
                        out0.at[pl.ds(NS * main_rows, tail_rows)])

        @pl.when(cid == 1)
        def _():
            pltpu.sync_copy(acc_sh.at[pl.ds(row0, main_rows)],
                            out1.at[pl.ds(row0, main_rows)])
            if tail_rows:
                @pl.when(sid == NS - 1)
                def _():
                    pltpu.sync_copy(
                        acc_sh.at[pl.ds(NS * main_rows, tail_rows)],
                        out1.at[pl.ds(NS * main_rows, tail_rows)])

    return sc_agg


def kernel(x, edge_index, edge_attr, W0, b0, bn_gamma, bn_beta, bn_mean,
           bn_var, W1, b1):
    n, din = x.shape
    e = edge_index.shape[1]
    h_dim = W0.shape[1]
    src = edge_index[0]
    dst = edge_index[1]
    zeros = jnp.zeros((n, h_dim), jnp.float32)

    chunk = 50
    src2d = src.reshape(e // chunk, chunk)
    dst2d = dst.reshape(e // chunk, chunk)
    attr2d = edge_attr.astype(jnp.float32).reshape(e // 32, 128)
    h, ew2d = _matmul_ew(x.astype(jnp.float32), W0, attr2d,
                         block_rows=n // 10)
    ew = ew2d.reshape(-1)
    agg_w = _make_sc_agg(True, n, e, h_dim, chunk)
    p0, p1 = agg_w(h, src2d, dst2d, ew, zeros)
    h1 = _bn_relu(p0, p1, b0, bn_gamma, bn_beta, bn_mean, bn_var)
    agg_p = _make_sc_agg(False, n, e, h_dim, chunk)
    q0, q1 = agg_p(h1, src2d, dst2d, ew, zeros)
    return _matmul_bias(q0, q1, W1, b1)


# speedup vs baseline: 6.1120x; 1.2322x over previous
import functools

import jax
import jax.numpy as jnp
from jax import lax
from jax.experimental import pallas as pl
from jax.experimental.pallas import tpu as pltpu
from jax.experimental.pallas import tpu_sc as plsc

NC = 2
NS = 16
LANES = 16
BN_EPS_ = 1e-5


def _mm_ew_body(x_ref, w_ref, a_ref, o_ref, ew_ref):
    o_ref[...] = jnp.dot(x_ref[...], w_ref[...],
                         preferred_element_type=jnp.float32)
    lane = lax.broadcasted_iota(jnp.int32, (128, 32), 0)
    grp = lax.broadcasted_iota(jnp.int32, (128, 32), 1)
    sel = jnp.where(lane // 4 == grp, 0.25, 0.0).astype(jnp.float32)
    ew_ref[...] = jnp.dot(a_ref[...], sel,
                          preferred_element_type=jnp.float32)


def _matmul_ew(x, w, attr2d, block_rows=1000):
    n, k = x.shape
    _, m = w.shape
    na = attr2d.shape[0]
    assert na == n
    grid = n // block_rows
    return pl.pallas_call(
        _mm_ew_body,
        grid=(grid,),
        in_specs=[
            pl.BlockSpec((block_rows, k), lambda i: (i, 0)),
            pl.BlockSpec((k, m), lambda i: (0, 0)),
            pl.BlockSpec((block_rows, 128), lambda i: (i, 0)),
        ],
        out_specs=[
            pl.BlockSpec((block_rows, m), lambda i: (i, 0)),
            pl.BlockSpec((block_rows, 32), lambda i: (i, 0)),
        ],
        out_shape=[
            jax.ShapeDtypeStruct((n, m), jnp.float32),
            jax.ShapeDtypeStruct((na, 32), jnp.float32),
        ],
    )(x, w, attr2d)


def _bn_relu_body(p0_ref, p1_ref, b_ref, g_ref, be_ref, mu_ref, var_ref,
                  o_ref):
    scale = g_ref[...] * lax.rsqrt(var_ref[...] + BN_EPS_)
    shift = be_ref[...] - mu_ref[...] * scale + b_ref[...] * scale
    agg = p0_ref[...] + p1_ref[...]
    o_ref[...] = jnp.maximum(agg * scale + shift, 0.0)


def _bn_relu(p0, p1, b0, gamma, beta, mean, var, block_rows=1000):
    n, d = p0.shape
    vec = lambda a: a.reshape(1, d)
    vspec = pl.BlockSpec((1, d), lambda i: (0, 0))
    bspec = pl.BlockSpec((block_rows, d), lambda i: (i, 0))
    return pl.pallas_call(
        _bn_relu_body,
        grid=(n // block_rows,),
        in_specs=[bspec, bspec, vspec, vspec, vspec, vspec, vspec],
        out_specs=bspec,
        out_shape=jax.ShapeDtypeStruct((n, d), jnp.float32),
    )(p0, p1, vec(b0), vec(gamma), vec(beta), vec(mean), vec(var))


def _mm_bias_body(q0_ref, q1_ref, w_ref, b_ref, o_ref):
    agg = q0_ref[...] + q1_ref[...]
    o_ref[...] = jnp.dot(agg, w_ref[...],
                         preferred_element_type=jnp.float32) + b_ref[...]


def _matmul_bias(q0, q1, w, b, block_rows=1000):
    n, k = q0.shape
    _, m = w.shape
    bspec = pl.BlockSpec((block_rows, k), lambda i: (i, 0))
    return pl.pallas_call(
        _mm_bias_body,
        grid=(n // block_rows,),
        in_specs=[
            bspec, bspec,
            pl.BlockSpec((k, m), lambda i: (0, 0)),
            pl.BlockSpec((1, m), lambda i: (0, 0)),
        ],
        out_specs=pl.BlockSpec((block_rows, m), lambda i: (i, 0)),
        out_shape=jax.ShapeDtypeStruct((n, m), jnp.float32),
    )(q0, q1, w, b.reshape(1, m))


def _make_sc_agg(weighted, n, e, d, chunk):
    per_tile = e // (NC * NS)
    assert per_tile * NC * NS == e
    n_chunks = per_tile // chunk
    n_pairs = n_chunks // 2
    assert n_chunks * chunk == per_tile and n_pairs * 2 == n_chunks
    assert chunk <= 128
    main_rows = (n // NS) // 8 * 8
    tail_rows = n - main_rows * NS
    assert 0 <= tail_rows and tail_rows % 8 == 0

    mesh = plsc.VectorSubcoreMesh(core_axis_name="c", subcore_axis_name="s")
    out_sds = jax.ShapeDtypeStruct((n, d), jnp.float32)

    scratch = [
        pltpu.VMEM_SHARED((n, d), jnp.float32),
        pltpu.VMEM((n_chunks, chunk), jnp.int32),
        pltpu.VMEM((n_chunks, chunk), jnp.int32),
        pltpu.VMEM((chunk, d), jnp.float32),
        pltpu.VMEM((chunk, d), jnp.float32),
        pltpu.SemaphoreType.DMA,
        pltpu.SemaphoreType.DMA,
        pltpu.SemaphoreType.DMA,
        pltpu.SemaphoreType.DMA,
    ]
    if weighted:
        scratch.append(pltpu.VMEM((per_tile + LANES,), jnp.float32))

    @functools.partial(
        pl.kernel, mesh=mesh,
        out_type=(out_sds, out_sds),
        scratch_types=scratch,
        compiler_params=pltpu.CompilerParams(use_tc_tiling_on_sc=False),
    )
    def sc_agg(h_hbm, src_hbm, dst_hbm, attr_hbm, zero_hbm,
               out0, out1, acc_sh, srcb, dstb, rows0, rows1,
               g0, g1, s0, s1, *maybe_ew):
        cid = lax.axis_index("c")
        sid = lax.axis_index("s")
        wid = sid * NC + cid
        row0 = pl.multiple_of(sid * main_rows, 8)
        rowsb = (rows0, rows1)
        gsem = (g0, g1)
        ssem = (s0, s1)

        pltpu.sync_copy(src_hbm.at[pl.ds(wid * n_chunks, n_chunks)], srcb)
        pltpu.sync_copy(dst_hbm.at[pl.ds(wid * n_chunks, n_chunks)], dstb)
        if weighted:
            ew_v = maybe_ew[0]
            pltpu.sync_copy(attr_hbm.at[pl.ds(wid * per_tile, per_tile)],
                            ew_v.at[pl.ds(0, per_tile)])

        pltpu.sync_copy(zero_hbm.at[pl.ds(row0, main_rows)],
                        acc_sh.at[pl.ds(row0, main_rows)])
        if tail_rows:
            @pl.when(sid == NS - 1)
            def _():
                pltpu.sync_copy(zero_hbm.at[pl.ds(NS * main_rows, tail_rows)],
                                acc_sh.at[pl.ds(NS * main_rows, tail_rows)])

        pltpu.async_copy(h_hbm.at[srcb.at[0]], rows0, g0)
        plsc.subcore_barrier()

        def scale_rows(ci, rv):
            def edge_body(ei, c2):
                w = ew_v[pl.ds(ci * chunk + ei, LANES)][0]
                for q in range(d // LANES):
                    sl = pl.ds(q * LANES, LANES)
                    rv[ei, sl] = rv[ei, sl] * w
                return c2
            lax.fori_loop(0, chunk, edge_body, 0)

        def pair_body(i, carry):
            for b in (0, 1):
                ci = 2 * i + b
                rv, nrv = rowsb[b], rowsb[1 - b]
                pltpu.make_async_copy(h_hbm.at[srcb.at[ci]], rv,
                                      gsem[b]).wait()
                if b == 0:
                    @pl.when(i >= 1)
                    def _():
                        pltpu.make_async_copy(
                            nrv, acc_sh.at[dstb.at[ci - 1]],
                            ssem[1 - b]).wait()
                    pltpu.async_copy(h_hbm.at[srcb.at[ci + 1]], nrv,
                                     gsem[1 - b])
                else:
                    pltpu.make_async_copy(nrv, acc_sh.at[dstb.at[ci - 1]],
                                          ssem[1 - b]).wait()

                    @pl.when(i < n_pairs - 1)
                    def _():
                        pltpu.async_copy(h_hbm.at[srcb.at[ci + 1]], nrv,
                                         gsem[1 - b])
                if weighted:
                    scale_rows(ci, rv)
                pltpu.async_copy(rv, acc_sh.at[dstb.at[ci]], ssem[b],
                                 add=True)
            return carry
        lax.fori_loop(0, n_pairs, pair_body, 0)
        pltpu.make_async_copy(rows1, acc_sh.at[dstb.at[n_chunks - 1]],
                              s1).wait()

        plsc.subcore_barrier()

        @pl.when(cid == 0)
        def _():
            pltpu.sync_copy(acc_sh.at[pl.ds(row0, main_rows)],
                            out0.at[pl.ds(row0, main_rows)])
            if tail_rows:
                @pl.when(sid == NS - 1)
                def _():
                    pltpu.sync_copy(
                        acc_sh.at[pl.ds(NS * main_rows, tail_rows)],
                        out0.at[pl.ds(NS * main_rows, tail_rows)])

        @pl.when(cid == 1)
        def _():
            pltpu.sync_copy(acc_sh.at[pl.ds(row0, main_rows)],
                            out1.at[pl.ds(row0, main_rows)])
            if tail_rows:
                @pl.when(sid == NS - 1)
                def _():
                    pltpu.sync_copy(
                        acc_sh.at[pl.ds(NS * main_rows, tail_rows)],
                        out1.at[pl.ds(NS * main_rows, tail_rows)])

    return sc_agg


def kernel(x, edge_index, edge_attr, W0, b0, bn_gamma, bn_beta, bn_mean,
           bn_var, W1, b1):
    n, din = x.shape
    e = edge_index.shape[1]
    h_dim = W0.shape[1]
    src = edge_index[0]
    dst = edge_index[1]
    zeros = jnp.zeros((n, h_dim), jnp.float32)

    chunk = 125
    src2d = src.reshape(e // chunk, chunk)
    dst2d = dst.reshape(e // chunk, chunk)
    attr2d = edge_attr.astype(jnp.float32).reshape(e // 32, 128)
    h, ew2d = _matmul_ew(x.astype(jnp.float32), W0, attr2d,
                         block_rows=n // 10)
    ew = ew2d.reshape(-1)
    agg_w = _make_sc_agg(True, n, e, h_dim, chunk)
    p0, p1 = agg_w(h, src2d, dst2d, ew, zeros)
    h1 = _bn_relu(p0, p1, b0, bn_gamma, bn_beta, bn_mean, bn_var)
    agg_p = _make_sc_agg(False, n, e, h_dim, chunk)
    q0, q1 = agg_p(h1, src2d, dst2d, ew, zeros)
    return _matmul_bias(q0, q1, W1, b1)


# speedup vs baseline: 6.1693x; 1.0094x over previous
import functools

import jax
import jax.numpy as jnp
from jax import lax
from jax.experimental import pallas as pl
from jax.experimental.pallas import tpu as pltpu
from jax.experimental.pallas import tpu_sc as plsc

NC = 2
NS = 16
LANES = 16
BN_EPS_ = 1e-5


def _mm_ew_body(x_ref, w_ref, a_ref, o_ref, ew_ref):
    o_ref[...] = jnp.dot(x_ref[...], w_ref[...],
                         preferred_element_type=jnp.float32)
    lane = lax.broadcasted_iota(jnp.int32, (128, 32), 0)
    grp = lax.broadcasted_iota(jnp.int32, (128, 32), 1)
    sel = jnp.where(lane // 4 == grp, 0.25, 0.0).astype(jnp.float32)
    ew_ref[...] = jnp.dot(a_ref[...], sel,
                          preferred_element_type=jnp.float32)


def _matmul_ew(x, w, attr2d, block_rows=1000):
    n, k = x.shape
    _, m = w.shape
    na = attr2d.shape[0]
    assert na == n
    grid = n // block_rows
    return pl.pallas_call(
        _mm_ew_body,
        grid=(grid,),
        in_specs=[
            pl.BlockSpec((block_rows, k), lambda i: (i, 0)),
            pl.BlockSpec((k, m), lambda i: (0, 0)),
            pl.BlockSpec((block_rows, 128), lambda i: (i, 0)),
        ],
        out_specs=[
            pl.BlockSpec((block_rows, m), lambda i: (i, 0)),
            pl.BlockSpec((block_rows, 32), lambda i: (i, 0)),
        ],
        out_shape=[
            jax.ShapeDtypeStruct((n, m), jnp.float32),
            jax.ShapeDtypeStruct((na, 32), jnp.float32),
        ],
    )(x, w, attr2d)


def _bn_relu_body(p0_ref, p1_ref, b_ref, g_ref, be_ref, mu_ref, var_ref,
                  o_ref):
    scale = g_ref[...] * lax.rsqrt(var_ref[...] + BN_EPS_)
    shift = be_ref[...] - mu_ref[...] * scale + b_ref[...] * scale
    agg = p0_ref[...] + p1_ref[...]
    o_ref[...] = jnp.maximum(agg * scale + shift, 0.0)


def _bn_relu(p0, p1, b0, gamma, beta, mean, var, block_rows=1000):
    n, d = p0.shape
    vec = lambda a: a.reshape(1, d)
    vspec = pl.BlockSpec((1, d), lambda i: (0, 0))
    bspec = pl.BlockSpec((block_rows, d), lambda i: (i, 0))
    return pl.pallas_call(
        _bn_relu_body,
        grid=(n // block_rows,),
        in_specs=[bspec, bspec, vspec, vspec, vspec, vspec, vspec],
        out_specs=bspec,
        out_shape=jax.ShapeDtypeStruct((n, d), jnp.float32),
    )(p0, p1, vec(b0), vec(gamma), vec(beta), vec(mean), vec(var))


def _mm_bias_body(q0_ref, q1_ref, w_ref, b_ref, o_ref):
    agg = q0_ref[...] + q1_ref[...]
    o_ref[...] = jnp.dot(agg, w_ref[...],
                         preferred_element_type=jnp.float32) + b_ref[...]


def _matmul_bias(q0, q1, w, b, block_rows=1000):
    n, k = q0.shape
    _, m = w.shape
    bspec = pl.BlockSpec((block_rows, k), lambda i: (i, 0))
    return pl.pallas_call(
        _mm_bias_body,
        grid=(n // block_rows,),
        in_specs=[
            bspec, bspec,
            pl.BlockSpec((k, m), lambda i: (0, 0)),
            pl.BlockSpec((1, m), lambda i: (0, 0)),
        ],
        out_specs=pl.BlockSpec((block_rows, m), lambda i: (i, 0)),
        out_shape=jax.ShapeDtypeStruct((n, m), jnp.float32),
    )(q0, q1, w, b.reshape(1, m))


def _make_sc_agg(weighted, n, e, d, chunk):
    per_tile = e // (NC * NS)
    assert per_tile * NC * NS == e
    n_chunks = per_tile // chunk
    n_pairs = n_chunks // 2
    assert n_chunks * chunk == per_tile and n_pairs * 2 == n_chunks
    assert chunk <= 128
    main_rows = (n // NS) // 8 * 8
    tail_rows = n - main_rows * NS
    assert 0 <= tail_rows and tail_rows % 8 == 0

    mesh = plsc.VectorSubcoreMesh(core_axis_name="c", subcore_axis_name="s")
    out_sds = jax.ShapeDtypeStruct((n, d), jnp.float32)

    scratch = [
        pltpu.VMEM_SHARED((n, d), jnp.float32),
        pltpu.VMEM_SHARED((n, d), jnp.float32),
        pltpu.VMEM((n_chunks, chunk), jnp.int32),
        pltpu.VMEM((n_chunks, chunk), jnp.int32),
        pltpu.VMEM((chunk, d), jnp.float32),
        pltpu.VMEM((chunk, d), jnp.float32),
        pltpu.SemaphoreType.DMA,
        pltpu.SemaphoreType.DMA,
        pltpu.SemaphoreType.DMA,
        pltpu.SemaphoreType.DMA,
    ]
    if weighted:
        scratch.append(pltpu.VMEM((per_tile + LANES,), jnp.float32))

    @functools.partial(
        pl.kernel, mesh=mesh,
        out_type=(out_sds, out_sds),
        scratch_types=scratch,
        compiler_params=pltpu.CompilerParams(use_tc_tiling_on_sc=False),
    )
    def sc_agg(h_hbm, src_hbm, dst_hbm, attr_hbm, zero_hbm,
               out0, out1, acc_sh, h_sh, srcb, dstb, rows0, rows1,
               g0, g1, s0, s1, *maybe_ew):
        cid = lax.axis_index("c")
        sid = lax.axis_index("s")
        wid = sid * NC + cid
        row0 = pl.multiple_of(sid * main_rows, 8)
        rowsb = (rows0, rows1)
        gsem = (g0, g1)
        ssem = (s0, s1)

        pltpu.sync_copy(src_hbm.at[pl.ds(wid * n_chunks, n_chunks)], srcb)
        pltpu.sync_copy(dst_hbm.at[pl.ds(wid * n_chunks, n_chunks)], dstb)
        if weighted:
            ew_v = maybe_ew[0]
            pltpu.sync_copy(attr_hbm.at[pl.ds(wid * per_tile, per_tile)],
                            ew_v.at[pl.ds(0, per_tile)])

        pltpu.sync_copy(zero_hbm.at[pl.ds(row0, main_rows)],
                        acc_sh.at[pl.ds(row0, main_rows)])
        pltpu.sync_copy(h_hbm.at[pl.ds(row0, main_rows)],
                        h_sh.at[pl.ds(row0, main_rows)])
        if tail_rows:
            @pl.when(sid == NS - 1)
            def _():
                pltpu.sync_copy(zero_hbm.at[pl.ds(NS * main_rows, tail_rows)],
                                acc_sh.at[pl.ds(NS * main_rows, tail_rows)])
                pltpu.sync_copy(h_hbm.at[pl.ds(NS * main_rows, tail_rows)],
                                h_sh.at[pl.ds(NS * main_rows, tail_rows)])
        plsc.subcore_barrier()
        pltpu.async_copy(h_sh.at[srcb.at[0]], rows0, g0)

        def scale_rows(ci, rv):
            def edge_body(ei, c2):
                w = ew_v[pl.ds(ci * chunk + ei, LANES)][0]
                for q in range(d // LANES):
                    sl = pl.ds(q * LANES, LANES)
                    rv[ei, sl] = rv[ei, sl] * w
                return c2
            lax.fori_loop(0, chunk, edge_body, 0)

        def pair_body(i, carry):
            for b in (0, 1):
                ci = 2 * i + b
                rv, nrv = rowsb[b], rowsb[1 - b]
                pltpu.make_async_copy(h_sh.at[srcb.at[ci]], rv,
                                      gsem[b]).wait()
                if b == 0:
                    @pl.when(i >= 1)
                    def _():
                        pltpu.make_async_copy(
                            nrv, acc_sh.at[dstb.at[ci - 1]],
                            ssem[1 - b]).wait()
                    pltpu.async_copy(h_sh.at[srcb.at[ci + 1]], nrv,
                                     gsem[1 - b])
                else:
                    pltpu.make_async_copy(nrv, acc_sh.at[dstb.at[ci - 1]],
                                          ssem[1 - b]).wait()

                    @pl.when(i < n_pairs - 1)
                    def _():
                        pltpu.async_copy(h_sh.at[srcb.at[ci + 1]], nrv,
                                         gsem[1 - b])
                if weighted:
                    scale_rows(ci, rv)
                pltpu.async_copy(rv, acc_sh.at[dstb.at[ci]], ssem[b],
                                 add=True)
            return carry
        lax.fori_loop(0, n_pairs, pair_body, 0)
        pltpu.make_async_copy(rows1, acc_sh.at[dstb.at[n_chunks - 1]],
                              s1).wait()

        plsc.subcore_barrier()

        @pl.when(cid == 0)
        def _():
            pltpu.sync_copy(acc_sh.at[pl.ds(row0, main_rows)],
                            out0.at[pl.ds(row0, main_rows)])
            if tail_rows:
                @pl.when(sid == NS - 1)
                def _():
                    pltpu.sync_copy(
                        acc_sh.at[pl.ds(NS * main_rows, tail_rows)],
                        out0.at[pl.ds(NS * main_rows, tail_rows)])

        @pl.when(cid == 1)
        def _():
            pltpu.sync_copy(acc_sh.at[pl.ds(row0, main_rows)],
                            out1.at[pl.ds(row0, main_rows)])
            if tail_rows:
                @pl.when(sid == NS - 1)
                def _():
                    pltpu.sync_copy(
                        acc_sh.at[pl.ds(NS * main_rows, tail_rows)],
                        out1.at[pl.ds(NS * main_rows, tail_rows)])

    return sc_agg


def kernel(x, edge_index, edge_attr, W0, b0, bn_gamma, bn_beta, bn_mean,
           bn_var, W1, b1):
    n, din = x.shape
    e = edge_index.shape[1]
    h_dim = W0.shape[1]
    src = edge_index[0]
    dst = edge_index[1]
    zeros = jnp.zeros((n, h_dim), jnp.float32)

    chunk = 125
    src2d = src.reshape(e // chunk, chunk)
    dst2d = dst.reshape(e // chunk, chunk)
    attr2d = edge_attr.astype(jnp.float32).reshape(e // 32, 128)
    h, ew2d = _matmul_ew(x.astype(jnp.float32), W0, attr2d,
                         block_rows=n // 10)
    ew = ew2d.reshape(-1)
    agg_w = _make_sc_agg(True, n, e, h_dim, chunk)
    p0, p1 = agg_w(h, src2d, dst2d, ew, zeros)
    h1 = _bn_relu(p0, p1, b0, bn_gamma, bn_beta, bn_mean, bn_var)
    agg_p = _make_sc_agg(False, n, e, h_dim, chunk)
    q0, q1 = agg_p(h1, src2d, dst2d, ew, zeros)
    return _matmul_bias(q0, q1, W1, b1)


# speedup vs baseline: 6.1711x; 1.0003x over previous
import functools

import jax
import jax.numpy as jnp
from jax import lax
from jax.experimental import pallas as pl
from jax.experimental.pallas import tpu as pltpu
from jax.experimental.pallas import tpu_sc as plsc

NC = 2
NS = 16
LANES = 16
BN_EPS_ = 1e-5


def _mm_ew_body(x_ref, w_ref, a_ref, o_ref, ew_ref):
    o_ref[...] = jnp.dot(x_ref[...], w_ref[...],
                         preferred_element_type=jnp.float32)
    lane = lax.broadcasted_iota(jnp.int32, (128, 32), 0)
    grp = lax.broadcasted_iota(jnp.int32, (128, 32), 1)
    sel = jnp.where(lane // 4 == grp, 0.25, 0.0).astype(jnp.float32)
    ew_ref[...] = jnp.dot(a_ref[...], sel,
                          preferred_element_type=jnp.float32)


def _matmul_ew(x, w, attr2d, block_rows=1000):
    n, k = x.shape
    _, m = w.shape
    na = attr2d.shape[0]
    assert na == n
    grid = n // block_rows
    return pl.pallas_call(
        _mm_ew_body,
        grid=(grid,),
        in_specs=[
            pl.BlockSpec((block_rows, k), lambda i: (i, 0)),
            pl.BlockSpec((k, m), lambda i: (0, 0)),
            pl.BlockSpec((block_rows, 128), lambda i: (i, 0)),
        ],
        out_specs=[
            pl.BlockSpec((block_rows, m), lambda i: (i, 0)),
            pl.BlockSpec((block_rows, 32), lambda i: (i, 0)),
        ],
        out_shape=[
            jax.ShapeDtypeStruct((n, m), jnp.float32),
            jax.ShapeDtypeStruct((na, 32), jnp.float32),
        ],
    )(x, w, attr2d)


def _bn_relu_body(p0_ref, p1_ref, b_ref, g_ref, be_ref, mu_ref, var_ref,
                  o_ref):
    scale = g_ref[...] * lax.rsqrt(var_ref[...] + BN_EPS_)
    shift = be_ref[...] - mu_ref[...] * scale + b_ref[...] * scale
    agg = p0_ref[...] + p1_ref[...]
    o_ref[...] = jnp.maximum(agg * scale + shift, 0.0)


def _bn_relu(p0, p1, b0, gamma, beta, mean, var, block_rows=1000):
    n, d = p0.shape
    vec = lambda a: a.reshape(1, d)
    vspec = pl.BlockSpec((1, d), lambda i: (0, 0))
    bspec = pl.BlockSpec((block_rows, d), lambda i: (i, 0))
    return pl.pallas_call(
        _bn_relu_body,
        grid=(n // block_rows,),
        in_specs=[bspec, bspec, vspec, vspec, vspec, vspec, vspec],
        out_specs=bspec,
        out_shape=jax.ShapeDtypeStruct((n, d), jnp.float32),
    )(p0, p1, vec(b0), vec(gamma), vec(beta), vec(mean), vec(var))


def _mm_bias_body(q0_ref, q1_ref, w_ref, b_ref, o_ref):
    agg = q0_ref[...] + q1_ref[...]
    o_ref[...] = jnp.dot(agg, w_ref[...],
                         preferred_element_type=jnp.float32) + b_ref[...]


def _matmul_bias(q0, q1, w, b, block_rows=1000):
    n, k = q0.shape
    _, m = w.shape
    bspec = pl.BlockSpec((block_rows, k), lambda i: (i, 0))
    return pl.pallas_call(
        _mm_bias_body,
        grid=(n // block_rows,),
        in_specs=[
            bspec, bspec,
            pl.BlockSpec((k, m), lambda i: (0, 0)),
            pl.BlockSpec((1, m), lambda i: (0, 0)),
        ],
        out_specs=pl.BlockSpec((block_rows, m), lambda i: (i, 0)),
        out_shape=jax.ShapeDtypeStruct((n, m), jnp.float32),
    )(q0, q1, w, b.reshape(1, m))


def _make_sc_agg(weighted, n, e, d, chunk):
    per_tile = e // (NC * NS)
    assert per_tile * NC * NS == e
    n_chunks = per_tile // chunk
    n_pairs = n_chunks // 2
    assert n_chunks * chunk == per_tile and n_pairs * 2 == n_chunks
    assert chunk <= 128
    main_rows = (n // NS) // 8 * 8
    tail_rows = n - main_rows * NS
    assert 0 <= tail_rows and tail_rows % 8 == 0

    mesh = plsc.VectorSubcoreMesh(core_axis_name="c", subcore_axis_name="s")
    out_sds = jax.ShapeDtypeStruct((n, d), jnp.float32)

    scratch = [
        pltpu.VMEM_SHARED((n, d), jnp.float32),
        pltpu.VMEM_SHARED((n, d), jnp.float32),
        pltpu.VMEM((n_chunks, chunk), jnp.int32),
        pltpu.VMEM((n_chunks, chunk), jnp.int32),
        pltpu.VMEM((chunk, d), jnp.float32),
        pltpu.VMEM((chunk, d), jnp.float32),
        pltpu.SemaphoreType.DMA,
        pltpu.SemaphoreType.DMA,
        pltpu.SemaphoreType.DMA,
        pltpu.SemaphoreType.DMA,
    ]
    if weighted:
        scratch.append(pltpu.VMEM((per_tile + LANES,), jnp.float32))

    @functools.partial(
        pl.kernel, mesh=mesh,
        out_type=(out_sds, out_sds),
        scratch_types=scratch,
        compiler_params=pltpu.CompilerParams(use_tc_tiling_on_sc=False),
    )
    def sc_agg(h_hbm, src_hbm, dst_hbm, attr_hbm, zero_hbm,
               out0, out1, acc_sh, h_sh, srcb, dstb, rows0, rows1,
               g0, g1, s0, s1, *maybe_ew):
        cid = lax.axis_index("c")
        sid = lax.axis_index("s")
        wid = sid * NC + cid
        row0 = pl.multiple_of(sid * main_rows, 8)
        rowsb = (rows0, rows1)
        gsem = (g0, g1)
        ssem = (s0, s1)

        pltpu.sync_copy(src_hbm.at[pl.ds(wid * n_chunks, n_chunks)], srcb)
        pltpu.sync_copy(dst_hbm.at[pl.ds(wid * n_chunks, n_chunks)], dstb)
        if weighted:
            ew_v = maybe_ew[0]
            pltpu.sync_copy(attr_hbm.at[pl.ds(wid * per_tile, per_tile)],
                            ew_v.at[pl.ds(0, per_tile)])

        pltpu.sync_copy(zero_hbm.at[pl.ds(row0, main_rows)],
                        acc_sh.at[pl.ds(row0, main_rows)])
        pltpu.sync_copy(h_hbm.at[pl.ds(row0, main_rows)],
                        h_sh.at[pl.ds(row0, main_rows)])
        if tail_rows:
            @pl.when(sid == NS - 1)
            def _():
                pltpu.sync_copy(zero_hbm.at[pl.ds(NS * main_rows, tail_rows)],
                                acc_sh.at[pl.ds(NS * main_rows, tail_rows)])
                pltpu.sync_copy(h_hbm.at[pl.ds(NS * main_rows, tail_rows)],
                                h_sh.at[pl.ds(NS * main_rows, tail_rows)])
        plsc.subcore_barrier()
        pltpu.async_copy(h_sh.at[srcb.at[0]], rows0, g0)

        def scale_rows(ci, rv):
            def edge_body(ei, c2):
                w = ew_v[pl.ds(ci * chunk + ei, LANES)][0]
                for q in range(d // LANES):
                    sl = pl.ds(q * LANES, LANES)
                    rv[ei, sl] = rv[ei, sl] * w
                return c2
            lax.fori_loop(0, chunk, edge_body, 0)

        def pair_body(i, carry):
            for b in (0, 1):
                ci = 2 * i + b
                rv, nrv = rowsb[b], rowsb[1 - b]
                pltpu.make_async_copy(h_sh.at[srcb.at[ci]], rv,
                                      gsem[b]).wait()
                if b == 0:
                    @pl.when(i >= 1)
                    def _():
                        pltpu.make_async_copy(
                            nrv, acc_sh.at[dstb.at[ci - 1]],
                            ssem[1 - b]).wait()
                    pltpu.async_copy(h_sh.at[srcb.at[ci + 1]], nrv,
                                     gsem[1 - b])
                else:
                    pltpu.make_async_copy(nrv, acc_sh.at[dstb.at[ci - 1]],
                                          ssem[1 - b]).wait()

                    @pl.when(i < n_pairs - 1)
                    def _():
                        pltpu.async_copy(h_sh.at[srcb.at[ci + 1]], nrv,
                                         gsem[1 - b])
                if weighted:
                    scale_rows(ci, rv)
                pltpu.async_copy(rv, acc_sh.at[dstb.at[ci]], ssem[b],
                                 add=True)
            return carry
        lax.fori_loop(0, n_pairs, pair_body, 0)
        pltpu.make_async_copy(rows1, acc_sh.at[dstb.at[n_chunks - 1]],
                              s1).wait()

        plsc.subcore_barrier()

        @pl.when(cid == 0)
        def _():
            pltpu.sync_copy(acc_sh.at[pl.ds(row0, main_rows)],
                            out0.at[pl.ds(row0, main_rows)])
            if tail_rows:
                @pl.when(sid == NS - 1)
                def _():
                    pltpu.sync_copy(
                        acc_sh.at[pl.ds(NS * main_rows, tail_rows)],
                        out0.at[pl.ds(NS * main_rows, tail_rows)])

        @pl.when(cid == 1)
        def _():
            pltpu.sync_copy(acc_sh.at[pl.ds(row0, main_rows)],
                            out1.at[pl.ds(row0, main_rows)])
            if tail_rows:
                @pl.when(sid == NS - 1)
                def _():
                    pltpu.sync_copy(
                        acc_sh.at[pl.ds(NS * main_rows, tail_rows)],
                        out1.at[pl.ds(NS * main_rows, tail_rows)])

    return sc_agg


def kernel(x, edge_index, edge_attr, W0, b0, bn_gamma, bn_beta, bn_mean,
           bn_var, W1, b1):
    n, din = x.shape
    e = edge_index.shape[1]
    h_dim = W0.shape[1]
    src = edge_index[0]
    dst = edge_index[1]
    zeros = jnp.zeros((n, h_dim), jnp.float32)

    chunk = 125
    src2d = src.reshape(e // chunk, chunk)
    dst2d = dst.reshape(e // chunk, chunk)
    attr2d = edge_attr.reshape(e // 32, 128)
    h, ew2d = _matmul_ew(x, W0, attr2d, block_rows=n // 10)
    ew = ew2d.reshape(-1)
    agg_w = _make_sc_agg(True, n, e, h_dim, chunk)
    p0, p1 = agg_w(h, src2d, dst2d, ew, zeros)
    h1 = _bn_relu(p0, p1, b0, bn_gamma, bn_beta, bn_mean, bn_var)
    agg_p = _make_sc_agg(False, n, e, h_dim, chunk)
    q0, q1 = agg_p(h1, src2d, dst2d, ew, zeros)
    return _matmul_bias(q0, q1, W1, b1)


# speedup vs baseline: 9.9060x; 1.6052x over previous
import functools

import jax
import jax.numpy as jnp
from jax import lax
from jax.experimental import pallas as pl
from jax.experimental.pallas import tpu as pltpu
from jax.experimental.pallas import tpu_sc as plsc

NC = 2
NS = 16
LANES = 16
BN_EPS_ = 1e-5


def _mm_body(x_ref, w_ref, o_ref):
    o_ref[...] = jnp.dot(x_ref[...], w_ref[...],
                         preferred_element_type=jnp.float32)


def _matmul(x, w, block_rows=1000):
    n, k = x.shape
    _, m = w.shape
    return pl.pallas_call(
        _mm_body,
        grid=(n // block_rows,),
        in_specs=[
            pl.BlockSpec((block_rows, k), lambda i: (i, 0)),
            pl.BlockSpec((k, m), lambda i: (0, 0)),
        ],
        out_specs=pl.BlockSpec((block_rows, m), lambda i: (i, 0)),
        out_shape=jax.ShapeDtypeStruct((n, m), jnp.float32),
    )(x, w)


def _ew_body(a0_ref, a1_ref, a2_ref, a3_ref, ew_ref):
    ew_ref[...] = (a0_ref[...] + a1_ref[...]
                   + a2_ref[...] + a3_ref[...]) * 0.25


def _edge_weights(attrs):
    na, nl = attrs[0].shape
    return pl.pallas_call(
        _ew_body,
        out_shape=jax.ShapeDtypeStruct((na, nl), jnp.float32),
    )(*attrs)


def _bn_relu_body(p0_ref, p1_ref, b_ref, g_ref, be_ref, mu_ref, var_ref,
                  o_ref):
    scale = g_ref[...] * lax.rsqrt(var_ref[...] + BN_EPS_)
    shift = be_ref[...] - mu_ref[...] * scale + b_ref[...] * scale
    agg = p0_ref[...] + p1_ref[...]
    o_ref[...] = jnp.maximum(agg * scale + shift, 0.0)


def _bn_relu(p0, p1, b0, gamma, beta, mean, var, block_rows=1000):
    n, d = p0.shape
    vec = lambda a: a.reshape(1, d)
    vspec = pl.BlockSpec((1, d), lambda i: (0, 0))
    bspec = pl.BlockSpec((block_rows, d), lambda i: (i, 0))
    return pl.pallas_call(
        _bn_relu_body,
        grid=(n // block_rows,),
        in_specs=[bspec, bspec, vspec, vspec, vspec, vspec, vspec],
        out_specs=bspec,
        out_shape=jax.ShapeDtypeStruct((n, d), jnp.float32),
    )(p0, p1, vec(b0), vec(gamma), vec(beta), vec(mean), vec(var))


def _mm_bias_body(q0_ref, q1_ref, w_ref, b_ref, o_ref):
    agg = q0_ref[...] + q1_ref[...]
    o_ref[...] = jnp.dot(agg, w_ref[...],
                         preferred_element_type=jnp.float32) + b_ref[...]


def _matmul_bias(q0, q1, w, b, block_rows=1000):
    n, k = q0.shape
    _, m = w.shape
    bspec = pl.BlockSpec((block_rows, k), lambda i: (i, 0))
    return pl.pallas_call(
        _mm_bias_body,
        grid=(n // block_rows,),
        in_specs=[
            bspec, bspec,
            pl.BlockSpec((k, m), lambda i: (0, 0)),
            pl.BlockSpec((1, m), lambda i: (0, 0)),
        ],
        out_specs=pl.BlockSpec((block_rows, m), lambda i: (i, 0)),
        out_shape=jax.ShapeDtypeStruct((n, m), jnp.float32),
    )(q0, q1, w, b.reshape(1, m))


def _make_sc_agg(weighted, n, e, d, chunk):
    per_tile = e // (NC * NS)
    assert per_tile * NC * NS == e
    n_chunks = per_tile // chunk
    n_pairs = n_chunks // 2
    assert n_chunks * chunk == per_tile and n_pairs * 2 == n_chunks
    assert chunk <= 128
    main_rows = (n // NS) // 8 * 8
    tail_rows = n - main_rows * NS
    assert 0 <= tail_rows and tail_rows % 8 == 0

    mesh = plsc.VectorSubcoreMesh(core_axis_name="c", subcore_axis_name="s")
    out_sds = jax.ShapeDtypeStruct((n, d), jnp.float32)

    scratch = [
        pltpu.VMEM_SHARED((n, d), jnp.float32),
        pltpu.VMEM_SHARED((n, d), jnp.float32),
        pltpu.VMEM((n_chunks, chunk), jnp.int32),
        pltpu.VMEM((n_chunks, chunk), jnp.int32),
        pltpu.VMEM((chunk, d), jnp.float32),
        pltpu.VMEM((chunk, d), jnp.float32),
        pltpu.SemaphoreType.DMA,
        pltpu.SemaphoreType.DMA,
        pltpu.SemaphoreType.DMA,
        pltpu.SemaphoreType.DMA,
    ]
    if weighted:
        scratch.append(pltpu.VMEM((per_tile + LANES,), jnp.float32))

    @functools.partial(
        pl.kernel, mesh=mesh,
        out_type=(out_sds, out_sds),
        scratch_types=scratch,
        compiler_params=pltpu.CompilerParams(use_tc_tiling_on_sc=False),
    )
    def sc_agg(h_hbm, src_hbm, dst_hbm, attr_hbm, zero_hbm,
               out0, out1, acc_sh, h_sh, srcb, dstb, rows0, rows1,
               g0, g1, s0, s1, *maybe_ew):
        cid = lax.axis_index("c")
        sid = lax.axis_index("s")
        wid = sid * NC + cid
        row0 = pl.multiple_of(sid * main_rows, 8)
        rowsb = (rows0, rows1)
        gsem = (g0, g1)
        ssem = (s0, s1)

        pltpu.sync_copy(src_hbm.at[pl.ds(wid * n_chunks, n_chunks)], srcb)
        pltpu.sync_copy(dst_hbm.at[pl.ds(wid * n_chunks, n_chunks)], dstb)
        if weighted:
            ew_v = maybe_ew[0]
            pltpu.sync_copy(attr_hbm.at[pl.ds(wid * per_tile, per_tile)],
                            ew_v.at[pl.ds(0, per_tile)])

        pltpu.sync_copy(zero_hbm.at[pl.ds(row0, main_rows)],
                        acc_sh.at[pl.ds(row0, main_rows)])
        pltpu.sync_copy(h_hbm.at[pl.ds(row0, main_rows)],
                        h_sh.at[pl.ds(row0, main_rows)])
        if tail_rows:
            @pl.when(sid == NS - 1)
            def _():
                pltpu.sync_copy(zero_hbm.at[pl.ds(NS * main_rows, tail_rows)],
                                acc_sh.at[pl.ds(NS * main_rows, tail_rows)])
                pltpu.sync_copy(h_hbm.at[pl.ds(NS * main_rows, tail_rows)],
                                h_sh.at[pl.ds(NS * main_rows, tail_rows)])
        plsc.subcore_barrier()
        pltpu.async_copy(h_sh.at[srcb.at[0]], rows0, g0)

        def scale_rows(ci, rv):
            def edge_body(ei, c2):
                w = ew_v[pl.ds(ci * chunk + ei, LANES)][0]
                for q in range(d // LANES):
                    sl = pl.ds(q * LANES, LANES)
                    rv[ei, sl] = rv[ei, sl] * w
                return c2
            lax.fori_loop(0, chunk, edge_body, 0)

        def pair_body(i, carry):
            for b in (0, 1):
                ci = 2 * i + b
                rv, nrv = rowsb[b], rowsb[1 - b]
                pltpu.make_async_copy(h_sh.at[srcb.at[ci]], rv,
                                      gsem[b]).wait()
                if b == 0:
                    @pl.when(i >= 1)
                    def _():
                        pltpu.make_async_copy(
                            nrv, acc_sh.at[dstb.at[ci - 1]],
                            ssem[1 - b]).wait()
                    pltpu.async_copy(h_sh.at[srcb.at[ci + 1]], nrv,
                                     gsem[1 - b])
                else:
                    pltpu.make_async_copy(nrv, acc_sh.at[dstb.at[ci - 1]],
                                          ssem[1 - b]).wait()

                    @pl.when(i < n_pairs - 1)
                    def _():
                        pltpu.async_copy(h_sh.at[srcb.at[ci + 1]], nrv,
                                         gsem[1 - b])
                if weighted:
                    scale_rows(ci, rv)
                pltpu.async_copy(rv, acc_sh.at[dstb.at[ci]], ssem[b],
                                 add=True)
            return carry
        lax.fori_loop(0, n_pairs, pair_body, 0)
        pltpu.make_async_copy(rows1, acc_sh.at[dstb.at[n_chunks - 1]],
                              s1).wait()

        plsc.subcore_barrier()

        @pl.when(cid == 0)
        def _():
            pltpu.sync_copy(acc_sh.at[pl.ds(row0, main_rows)],
                            out0.at[pl.ds(row0, main_rows)])
            if tail_rows:
                @pl.when(sid == NS - 1)
                def _():
                    pltpu.sync_copy(
                        acc_sh.at[pl.ds(NS * main_rows, tail_rows)],
                        out0.at[pl.ds(NS * main_rows, tail_rows)])

        @pl.when(cid == 1)
        def _():
            pltpu.sync_copy(acc_sh.at[pl.ds(row0, main_rows)],
                            out1.at[pl.ds(row0, main_rows)])
            if tail_rows:
                @pl.when(sid == NS - 1)
                def _():
                    pltpu.sync_copy(
                        acc_sh.at[pl.ds(NS * main_rows, tail_rows)],
                        out1.at[pl.ds(NS * main_rows, tail_rows)])

    return sc_agg


def kernel(x, edge_index, edge_attr, W0, b0, bn_gamma, bn_beta, bn_mean,
           bn_var, W1, b1):
    n, din = x.shape
    e = edge_index.shape[1]
    h_dim = W0.shape[1]
    src = edge_index[0]
    dst = edge_index[1]
    zeros = jnp.zeros((n, h_dim), jnp.float32)

    chunk = 125
    src2d = src.reshape(e // chunk, chunk)
    dst2d = dst.reshape(e // chunk, chunk)
    attrs = [edge_attr[:, i].reshape(e // 128, 128) for i in range(4)]
    h = _matmul(x, W0, block_rows=n // 10)
    ew = _edge_weights(attrs).reshape(-1)
    agg_w = _make_sc_agg(True, n, e, h_dim, chunk)
    p0, p1 = agg_w(h, src2d, dst2d, ew, zeros)
    h1 = _bn_relu(p0, p1, b0, bn_gamma, bn_beta, bn_mean, bn_var)
    agg_p = _make_sc_agg(False, n, e, h_dim, chunk)
    q0, q1 = agg_p(h1, src2d, dst2d, ew, zeros)
    return _matmul_bias(q0, q1, W1, b1)


# speedup vs baseline: 10.9142x; 1.1018x over previous
import functools

import jax
import jax.numpy as jnp
from jax import lax
from jax.experimental import pallas as pl
from jax.experimental.pallas import tpu as pltpu
from jax.experimental.pallas import tpu_sc as plsc

NC = 2
NS = 16
LANES = 16
BN_EPS_ = 1e-5


def _mm_body(x_ref, w_ref, o_ref):
    o_ref[...] = jnp.dot(x_ref[...], w_ref[...],
                         preferred_element_type=jnp.float32)


def _matmul(x, w, block_rows=1000):
    n, k = x.shape
    _, m = w.shape
    return pl.pallas_call(
        _mm_body,
        grid=(n // block_rows,),
        in_specs=[
            pl.BlockSpec((block_rows, k), lambda i: (i, 0)),
            pl.BlockSpec((k, m), lambda i: (0, 0)),
        ],
        out_specs=pl.BlockSpec((block_rows, m), lambda i: (i, 0)),
        out_shape=jax.ShapeDtypeStruct((n, m), jnp.float32),
    )(x, w)


def _ew_body(a0_ref, a1_ref, a2_ref, a3_ref, b_ref, g_ref, be_ref, mu_ref,
             var_ref, ew_ref, sc_ref, sh_ref):
    ew_ref[...] = (a0_ref[...] + a1_ref[...]
                   + a2_ref[...] + a3_ref[...]) * 0.25
    scale = g_ref[...] * lax.rsqrt(var_ref[...] + BN_EPS_)
    sc_ref[...] = scale
    sh_ref[...] = be_ref[...] + (b_ref[...] - mu_ref[...]) * scale


def _edge_weights(attrs, b0, gamma, beta, mean, var):
    na, nl = attrs[0].shape
    d = b0.shape[0]
    vec = lambda a: a.reshape(1, d)
    vsds = jax.ShapeDtypeStruct((1, d), jnp.float32)
    return pl.pallas_call(
        _ew_body,
        out_shape=[jax.ShapeDtypeStruct((na, nl), jnp.float32), vsds, vsds],
    )(*attrs, vec(b0), vec(gamma), vec(beta), vec(mean), vec(var))


def _bn_relu_body(p0_ref, p1_ref, b_ref, g_ref, be_ref, mu_ref, var_ref,
                  o_ref):
    scale = g_ref[...] * lax.rsqrt(var_ref[...] + BN_EPS_)
    shift = be_ref[...] - mu_ref[...] * scale + b_ref[...] * scale
    agg = p0_ref[...] + p1_ref[...]
    o_ref[...] = jnp.maximum(agg * scale + shift, 0.0)


def _bn_relu(p0, p1, b0, gamma, beta, mean, var, block_rows=1000):
    n, d = p0.shape
    vec = lambda a: a.reshape(1, d)
    vspec = pl.BlockSpec((1, d), lambda i: (0, 0))
    bspec = pl.BlockSpec((block_rows, d), lambda i: (i, 0))
    return pl.pallas_call(
        _bn_relu_body,
        grid=(n // block_rows,),
        in_specs=[bspec, bspec, vspec, vspec, vspec, vspec, vspec],
        out_specs=bspec,
        out_shape=jax.ShapeDtypeStruct((n, d), jnp.float32),
    )(p0, p1, vec(b0), vec(gamma), vec(beta), vec(mean), vec(var))


def _mm_bias_body(q0_ref, q1_ref, w_ref, b_ref, o_ref):
    agg = q0_ref[...] + q1_ref[...]
    o_ref[...] = jnp.dot(agg, w_ref[...],
                         preferred_element_type=jnp.float32) + b_ref[...]


def _matmul_bias(q0, q1, w, b, block_rows=1000):
    n, k = q0.shape
    _, m = w.shape
    bspec = pl.BlockSpec((block_rows, k), lambda i: (i, 0))
    return pl.pallas_call(
        _mm_bias_body,
        grid=(n // block_rows,),
        in_specs=[
            bspec, bspec,
            pl.BlockSpec((k, m), lambda i: (0, 0)),
            pl.BlockSpec((1, m), lambda i: (0, 0)),
        ],
        out_specs=pl.BlockSpec((block_rows, m), lambda i: (i, 0)),
        out_shape=jax.ShapeDtypeStruct((n, m), jnp.float32),
    )(q0, q1, w, b.reshape(1, m))


def _make_sc_agg(weighted, n, e, d, chunk, bn=False):
    assert not (weighted and bn)
    per_tile = e // (NC * NS)
    assert per_tile * NC * NS == e
    n_chunks = per_tile // chunk
    n_pairs = n_chunks // 2
    assert n_chunks * chunk == per_tile and n_pairs * 2 == n_chunks
    assert chunk <= 128
    main_rows = (n // NS) // 8 * 8
    tail_rows = n - main_rows * NS
    assert 0 <= tail_rows and tail_rows % 8 == 0

    mesh = plsc.VectorSubcoreMesh(core_axis_name="c", subcore_axis_name="s")
    out_sds = jax.ShapeDtypeStruct((n, d), jnp.float32)

    slab = main_rows // 8
    scratch = [
        pltpu.VMEM_SHARED((n, d), jnp.float32),
        pltpu.VMEM_SHARED((n, d), jnp.float32),
        pltpu.VMEM((n_chunks, chunk), jnp.int32),
        pltpu.VMEM((n_chunks, chunk), jnp.int32),
        pltpu.VMEM((chunk, d), jnp.float32),
        pltpu.VMEM((chunk, d), jnp.float32),
        pltpu.SemaphoreType.DMA,
        pltpu.SemaphoreType.DMA,
        pltpu.SemaphoreType.DMA,
        pltpu.SemaphoreType.DMA,
    ]
    if weighted:
        scratch.append(pltpu.VMEM((per_tile + LANES,), jnp.float32))
    if bn:
        scratch.extend([
            pltpu.VMEM((d,), jnp.float32),
            pltpu.VMEM((d,), jnp.float32),
            pltpu.VMEM((slab, d), jnp.float32),
            pltpu.VMEM((slab, d), jnp.float32),
        ])

    @functools.partial(
        pl.kernel, mesh=mesh,
        out_type=(out_sds, out_sds),
        scratch_types=scratch,
        compiler_params=pltpu.CompilerParams(use_tc_tiling_on_sc=False),
    )
    def sc_agg(*refs):
        if bn:
            (p0_hbm, p1_hbm, sc_hbm, sh_hbm, src_hbm, dst_hbm, zero_hbm,
             out0, out1, acc_sh, h_sh, srcb, dstb, rows0, rows1,
             g0, g1, s0, s1, sv, tv, pA, pB) = refs
        elif weighted:
            (h_hbm, src_hbm, dst_hbm, ew_hbm, zero_hbm,
             out0, out1, acc_sh, h_sh, srcb, dstb, rows0, rows1,
             g0, g1, s0, s1, ew_v) = refs
        else:
            (h_hbm, src_hbm, dst_hbm, zero_hbm,
             out0, out1, acc_sh, h_sh, srcb, dstb, rows0, rows1,
             g0, g1, s0, s1) = refs
        cid = lax.axis_index("c")
        sid = lax.axis_index("s")
        wid = sid * NC + cid
        row0 = pl.multiple_of(sid * main_rows, 8)
        rowsb = (rows0, rows1)
        gsem = (g0, g1)
        ssem = (s0, s1)

        pltpu.sync_copy(src_hbm.at[pl.ds(wid * n_chunks, n_chunks)], srcb)
        pltpu.sync_copy(dst_hbm.at[pl.ds(wid * n_chunks, n_chunks)], dstb)
        if weighted:
            pltpu.sync_copy(ew_hbm.at[pl.ds(wid * per_tile, per_tile)],
                            ew_v.at[pl.ds(0, per_tile)])

        pltpu.sync_copy(zero_hbm.at[pl.ds(row0, main_rows)],
                        acc_sh.at[pl.ds(row0, main_rows)])

        def stage_bn(r0, nrows, bufa, bufb):
            pltpu.sync_copy(p0_hbm.at[pl.ds(r0, nrows)], bufa)
            pltpu.sync_copy(p1_hbm.at[pl.ds(r0, nrows)], bufb)

            def rbody(ri, c2):
                for q in range(d // LANES):
                    sl = pl.ds(q * LANES, LANES)
                    v = (bufa[ri, sl] + bufb[ri, sl]) * sv[sl] + tv[sl]
                    bufa[ri, sl] = jnp.maximum(v, 0.0)
                return c2
            lax.fori_loop(0, nrows, rbody, 0)
            pltpu.sync_copy(bufa, h_sh.at[pl.ds(r0, nrows)])

        if bn:
            pltpu.sync_copy(sc_hbm, sv)
            pltpu.sync_copy(sh_hbm, tv)
            for k in range(main_rows // slab):
                stage_bn(row0 + k * slab, slab, pA, pB)
        else:
            pltpu.sync_copy(h_hbm.at[pl.ds(row0, main_rows)],
                            h_sh.at[pl.ds(row0, main_rows)])
        if tail_rows:
            @pl.when(sid == NS - 1)
            def _():
                pltpu.sync_copy(zero_hbm.at[pl.ds(NS * main_rows, tail_rows)],
                                acc_sh.at[pl.ds(NS * main_rows, tail_rows)])
                if bn:
                    stage_bn(NS * main_rows, tail_rows,
                             pA.at[pl.ds(0, tail_rows)],
                             pB.at[pl.ds(0, tail_rows)])
                else:
                    pltpu.sync_copy(
                        h_hbm.at[pl.ds(NS * main_rows, tail_rows)],
                        h_sh.at[pl.ds(NS * main_rows, tail_rows)])
        plsc.subcore_barrier()
        pltpu.async_copy(h_sh.at[srcb.at[0]], rows0, g0)

        grp = 10
        assert chunk % grp == 0

        def scale_rows(ci, rv):
            def grp_body(gi, c2):
                w16 = ew_v[pl.ds(ci * chunk + gi * grp, LANES)]
                for j in range(grp):
                    w = w16[j]
                    for q in range(d // LANES):
                        sl = pl.ds(q * LANES, LANES)
                        rv[gi * grp + j, sl] = rv[gi * grp + j, sl] * w
                return c2
            lax.fori_loop(0, chunk // grp, grp_body, 0)

        def pair_body(i, carry):
            for b in (0, 1):
                ci = 2 * i + b
                rv, nrv = rowsb[b], rowsb[1 - b]
                pltpu.make_async_copy(h_sh.at[srcb.at[ci]], rv,
                                      gsem[b]).wait()
                if b == 0:
                    @pl.when(i >= 1)
                    def _():
                        pltpu.make_async_copy(
                            nrv, acc_sh.at[dstb.at[ci - 1]],
                            ssem[1 - b]).wait()
                    pltpu.async_copy(h_sh.at[srcb.at[ci + 1]], nrv,
                                     gsem[1 - b])
                else:
                    pltpu.make_async_copy(nrv, acc_sh.at[dstb.at[ci - 1]],
                                          ssem[1 - b]).wait()

                    @pl.when(i < n_pairs - 1)
                    def _():
                        pltpu.async_copy(h_sh.at[srcb.at[ci + 1]], nrv,
                                         gsem[1 - b])
                if weighted:
                    scale_rows(ci, rv)
                pltpu.async_copy(rv, acc_sh.at[dstb.at[ci]], ssem[b],
                                 add=True)
            return carry
        lax.fori_loop(0, n_pairs, pair_body, 0)
        pltpu.make_async_copy(rows1, acc_sh.at[dstb.at[n_chunks - 1]],
                              s1).wait()

        plsc.subcore_barrier()

        @pl.when(cid == 0)
        def _():
            pltpu.sync_copy(acc_sh.at[pl.ds(row0, main_rows)],
                            out0.at[pl.ds(row0, main_rows)])
            if tail_rows:
                @pl.when(sid == NS - 1)
                def _():
                    pltpu.sync_copy(
                        acc_sh.at[pl.ds(NS * main_rows, tail_rows)],
                        out0.at[pl.ds(NS * main_rows, tail_rows)])

        @pl.when(cid == 1)
        def _():
            pltpu.sync_copy(acc_sh.at[pl.ds(row0, main_rows)],
                            out1.at[pl.ds(row0, main_rows)])
            if tail_rows:
                @pl.when(sid == NS - 1)
                def _():
                    pltpu.sync_copy(
                        acc_sh.at[pl.ds(NS * main_rows, tail_rows)],
                        out1.at[pl.ds(NS * main_rows, tail_rows)])

    return sc_agg


def kernel(x, edge_index, edge_attr, W0, b0, bn_gamma, bn_beta, bn_mean,
           bn_var, W1, b1):
    n, din = x.shape
    e = edge_index.shape[1]
    h_dim = W0.shape[1]
    src = edge_index[0]
    dst = edge_index[1]
    zeros = jnp.zeros((n, h_dim), jnp.float32)

    chunk = 100
    src2d = src.reshape(e // chunk, chunk)
    dst2d = dst.reshape(e // chunk, chunk)
    attrs = [edge_attr[:, i].reshape(e // 128, 128) for i in range(4)]
    h = _matmul(x, W0, block_rows=n // 10)
    ew2d, scale, shift = _edge_weights(attrs, b0, bn_gamma, bn_beta,
                                       bn_mean, bn_var)
    ew = ew2d.reshape(-1)
    agg_w = _make_sc_agg(True, n, e, h_dim, chunk)
    p0, p1 = agg_w(h, src2d, dst2d, ew, zeros)
    agg_p = _make_sc_agg(False, n, e, h_dim, chunk, bn=True)
    q0, q1 = agg_p(p0, p1, scale.reshape(-1), shift.reshape(-1),
                   src2d, dst2d, zeros)
    return _matmul_bias(q0, q1, W1, b1)


# speedup vs baseline: 10.9929x; 1.0072x over previous
import functools

import jax
import jax.numpy as jnp
from jax import lax
from jax.experimental import pallas as pl
from jax.experimental.pallas import tpu as pltpu
from jax.experimental.pallas import tpu_sc as plsc

NC = 2
NS = 16
LANES = 16
BN_EPS_ = 1e-5


def _mm_body(x_ref, w_ref, o_ref):
    o_ref[...] = jnp.dot(x_ref[...], w_ref[...],
                         preferred_element_type=jnp.float32)


def _matmul(x, w, block_rows=1000):
    n, k = x.shape
    _, m = w.shape
    return pl.pallas_call(
        _mm_body,
        grid=(n // block_rows,),
        in_specs=[
            pl.BlockSpec((block_rows, k), lambda i: (i, 0)),
            pl.BlockSpec((k, m), lambda i: (0, 0)),
        ],
        out_specs=pl.BlockSpec((block_rows, m), lambda i: (i, 0)),
        out_shape=jax.ShapeDtypeStruct((n, m), jnp.float32),
    )(x, w)


def _ew_body(a0_ref, a1_ref, a2_ref, a3_ref, b_ref, g_ref, be_ref, mu_ref,
             var_ref, ew_ref, sc_ref, sh_ref):
    ew_ref[...] = (a0_ref[...] + a1_ref[...]
                   + a2_ref[...] + a3_ref[...]) * 0.25
    scale = g_ref[...] * lax.rsqrt(var_ref[...] + BN_EPS_)
    sc_ref[...] = scale
    sh_ref[...] = be_ref[...] + (b_ref[...] - mu_ref[...]) * scale


def _edge_weights(attrs, b0, gamma, beta, mean, var):
    na, nl = attrs[0].shape
    d = b0.shape[0]
    vec = lambda a: a.reshape(1, d)
    vsds = jax.ShapeDtypeStruct((1, d), jnp.float32)
    return pl.pallas_call(
        _ew_body,
        out_shape=[jax.ShapeDtypeStruct((na, nl), jnp.float32), vsds, vsds],
    )(*attrs, vec(b0), vec(gamma), vec(beta), vec(mean), vec(var))


def _mm_bias_body(q0_ref, q1_ref, w_ref, b_ref, o_ref):
    agg = q0_ref[...] + q1_ref[...]
    o_ref[...] = jnp.dot(agg, w_ref[...],
                         preferred_element_type=jnp.float32) + b_ref[...]


def _matmul_bias(q0, q1, w, b, block_rows=1000):
    n, k = q0.shape
    _, m = w.shape
    bspec = pl.BlockSpec((block_rows, k), lambda i: (i, 0))
    return pl.pallas_call(
        _mm_bias_body,
        grid=(n // block_rows,),
        in_specs=[
            bspec, bspec,
            pl.BlockSpec((k, m), lambda i: (0, 0)),
            pl.BlockSpec((1, m), lambda i: (0, 0)),
        ],
        out_specs=pl.BlockSpec((block_rows, m), lambda i: (i, 0)),
        out_shape=jax.ShapeDtypeStruct((n, m), jnp.float32),
    )(q0, q1, w, b.reshape(1, m))


def _make_sc_agg(weighted, n, e, d, chunk, bn=False):
    assert not (weighted and bn)
    per_tile = e // (NC * NS)
    assert per_tile * NC * NS == e
    n_chunks = per_tile // chunk
    n_pairs = n_chunks // 2
    assert n_chunks * chunk == per_tile and n_pairs * 2 == n_chunks
    assert chunk <= 128
    main_rows = (n // NS) // 8 * 8
    tail_rows = n - main_rows * NS
    assert 0 <= tail_rows and tail_rows % 8 == 0

    mesh = plsc.VectorSubcoreMesh(core_axis_name="c", subcore_axis_name="s")
    out_sds = jax.ShapeDtypeStruct((n, d), jnp.float32)

    slab = main_rows // 8
    scratch = [
        pltpu.VMEM_SHARED((n, d), jnp.float32),
        pltpu.VMEM_SHARED((n, d), jnp.float32),
        pltpu.VMEM((n_chunks, chunk), jnp.int32),
        pltpu.VMEM((n_chunks, chunk), jnp.int32),
        pltpu.VMEM((chunk, d), jnp.float32),
        pltpu.VMEM((chunk, d), jnp.float32),
        pltpu.SemaphoreType.DMA,
        pltpu.SemaphoreType.DMA,
        pltpu.SemaphoreType.DMA,
        pltpu.SemaphoreType.DMA,
    ]
    if weighted:
        scratch.append(pltpu.VMEM((per_tile + LANES,), jnp.float32))
    if bn:
        scratch.extend([
            pltpu.VMEM((d,), jnp.float32),
            pltpu.VMEM((d,), jnp.float32),
            pltpu.VMEM((slab, d), jnp.float32),
            pltpu.VMEM((slab, d), jnp.float32),
        ])

    @functools.partial(
        pl.kernel, mesh=mesh,
        out_type=(out_sds, out_sds),
        scratch_types=scratch,
        compiler_params=pltpu.CompilerParams(use_tc_tiling_on_sc=False),
    )
    def sc_agg(*refs):
        if bn:
            (p0_hbm, p1_hbm, sc_hbm, sh_hbm, src_hbm, dst_hbm, zero_hbm,
             out0, out1, acc_sh, h_sh, srcb, dstb, rows0, rows1,
             g0, g1, s0, s1, sv, tv, pA, pB) = refs
        elif weighted:
            (h_hbm, src_hbm, dst_hbm, ew_hbm, zero_hbm,
             out0, out1, acc_sh, h_sh, srcb, dstb, rows0, rows1,
             g0, g1, s0, s1, ew_v) = refs
        else:
            (h_hbm, src_hbm, dst_hbm, zero_hbm,
             out0, out1, acc_sh, h_sh, srcb, dstb, rows0, rows1,
             g0, g1, s0, s1) = refs
        cid = lax.axis_index("c")
        sid = lax.axis_index("s")
        wid = sid * NC + cid
        row0 = pl.multiple_of(sid * main_rows, 8)
        rowsb = (rows0, rows1)
        gsem = (g0, g1)
        ssem = (s0, s1)

        pltpu.sync_copy(src_hbm.at[pl.ds(wid * n_chunks, n_chunks)], srcb)
        pltpu.sync_copy(dst_hbm.at[pl.ds(wid * n_chunks, n_chunks)], dstb)
        if weighted:
            pltpu.sync_copy(ew_hbm.at[pl.ds(wid * per_tile, per_tile)],
                            ew_v.at[pl.ds(0, per_tile)])

        pltpu.sync_copy(zero_hbm.at[pl.ds(row0, main_rows)],
                        acc_sh.at[pl.ds(row0, main_rows)])

        def stage_bn(r0, nrows, bufa, bufb):
            pltpu.sync_copy(p0_hbm.at[pl.ds(r0, nrows)], bufa)
            pltpu.sync_copy(p1_hbm.at[pl.ds(r0, nrows)], bufb)

            def rbody(ri, c2):
                for q in range(d // LANES):
                    sl = pl.ds(q * LANES, LANES)
                    v = (bufa[ri, sl] + bufb[ri, sl]) * sv[sl] + tv[sl]
                    bufa[ri, sl] = jnp.maximum(v, 0.0)
                return c2
            lax.fori_loop(0, nrows, rbody, 0)
            pltpu.sync_copy(bufa, h_sh.at[pl.ds(r0, nrows)])

        if bn:
            pltpu.sync_copy(sc_hbm, sv)
            pltpu.sync_copy(sh_hbm, tv)
            for k in range(main_rows // slab):
                stage_bn(row0 + k * slab, slab, pA, pB)
        else:
            pltpu.sync_copy(h_hbm.at[pl.ds(row0, main_rows)],
                            h_sh.at[pl.ds(row0, main_rows)])
        if tail_rows:
            @pl.when(sid == NS - 1)
            def _():
                pltpu.sync_copy(zero_hbm.at[pl.ds(NS * main_rows, tail_rows)],
                                acc_sh.at[pl.ds(NS * main_rows, tail_rows)])
                if bn:
                    stage_bn(NS * main_rows, tail_rows,
                             pA.at[pl.ds(0, tail_rows)],
                             pB.at[pl.ds(0, tail_rows)])
                else:
                    pltpu.sync_copy(
                        h_hbm.at[pl.ds(NS * main_rows, tail_rows)],
                        h_sh.at[pl.ds(NS * main_rows, tail_rows)])
        plsc.subcore_barrier()
        pltpu.async_copy(h_sh.at[srcb.at[0]], rows0, g0)

        grp = 10
        assert chunk % grp == 0

        def scale_rows(ci, rv):
            def grp_body(gi, c2):
                w16 = ew_v[pl.ds(ci * chunk + gi * grp, LANES)]
                for j in range(grp):
                    w = w16[j]
                    for q in range(d // LANES):
                        sl = pl.ds(q * LANES, LANES)
                        rv[gi * grp + j, sl] = rv[gi * grp + j, sl] * w
                return c2
            lax.fori_loop(0, chunk // grp, grp_body, 0)

        def pair_body(i, carry):
            for b in (0, 1):
                ci = 2 * i + b
                rv, nrv = rowsb[b], rowsb[1 - b]
                pltpu.make_async_copy(h_sh.at[srcb.at[ci]], rv,
                                      gsem[b]).wait()
                if b == 0:
                    @pl.when(i >= 1)
                    def _():
                        pltpu.make_async_copy(
                            nrv, acc_sh.at[dstb.at[ci - 1]],
                            ssem[1 - b]).wait()
                    pltpu.async_copy(h_sh.at[srcb.at[ci + 1]], nrv,
                                     gsem[1 - b])
                else:
                    pltpu.make_async_copy(nrv, acc_sh.at[dstb.at[ci - 1]],
                                          ssem[1 - b]).wait()

                    @pl.when(i < n_pairs - 1)
                    def _():
                        pltpu.async_copy(h_sh.at[srcb.at[ci + 1]], nrv,
                                         gsem[1 - b])
                if weighted:
                    scale_rows(ci, rv)
                pltpu.async_copy(rv, acc_sh.at[dstb.at[ci]], ssem[b],
                                 add=True)
            return carry
        lax.fori_loop(0, n_pairs, pair_body, 0)
        pltpu.make_async_copy(rows1, acc_sh.at[dstb.at[n_chunks - 1]],
                              s1).wait()

        plsc.subcore_barrier()

        @pl.when(cid == 0)
        def _():
            pltpu.sync_copy(acc_sh.at[pl.ds(row0, main_rows)],
                            out0.at[pl.ds(row0, main_rows)])
            if tail_rows:
                @pl.when(sid == NS - 1)
                def _():
                    pltpu.sync_copy(
                        acc_sh.at[pl.ds(NS * main_rows, tail_rows)],
                        out0.at[pl.ds(NS * main_rows, tail_rows)])

        @pl.when(cid == 1)
        def _():
            pltpu.sync_copy(acc_sh.at[pl.ds(row0, main_rows)],
                            out1.at[pl.ds(row0, main_rows)])
            if tail_rows:
                @pl.when(sid == NS - 1)
                def _():
                    pltpu.sync_copy(
                        acc_sh.at[pl.ds(NS * main_rows, tail_rows)],
                        out1.at[pl.ds(NS * main_rows, tail_rows)])

    return sc_agg


def kernel(x, edge_index, edge_attr, W0, b0, bn_gamma, bn_beta, bn_mean,
           bn_var, W1, b1):
    n, din = x.shape
    e = edge_index.shape[1]
    h_dim = W0.shape[1]
    src = edge_index[0]
    dst = edge_index[1]
    zeros = jnp.zeros((n, h_dim), jnp.float32)

    chunk = 100
    src2d = src.reshape(e // chunk, chunk)
    dst2d = dst.reshape(e // chunk, chunk)
    attrs = [edge_attr[:, i].reshape(e // 128, 128) for i in range(4)]
    h = _matmul(x, W0, block_rows=n // 10)
    ew2d, scale, shift = _edge_weights(attrs, b0, bn_gamma, bn_beta,
                                       bn_mean, bn_var)
    ew = ew2d.reshape(-1)
    agg_w = _make_sc_agg(True, n, e, h_dim, chunk)
    p0, p1 = agg_w(h, src2d, dst2d, ew, zeros)
    agg_p = _make_sc_agg(False, n, e, h_dim, chunk, bn=True)
    q0, q1 = agg_p(p0, p1, scale.reshape(-1), shift.reshape(-1),
                   src2d, dst2d, zeros)
    return _matmul_bias(q0, q1, W1, b1)
